# Initial kernel scaffold; baseline (speedup 1.0000x reference)
import jax
import jax.numpy as jnp
from jax.experimental import pallas as pl


def kernel(x, edge_index, batch, W1, b1, W2, b2, Wfc, bfc):
    raise NotImplementedError("write your pallas kernel here")


# speedup vs baseline: 16.3610x; 16.3610x over previous
import functools

import jax
import jax.numpy as jnp
from jax import lax
from jax.experimental import pallas as pl
from jax.experimental.pallas import tpu as pltpu
from jax.experimental.pallas import tpu_sc as plsc

_NC = 2
_NS = 16
_NW = _NC * _NS
_CH = 100
_LANES = 16


def _row_partition(n_nodes):
    rq = 8 * (n_nodes // (8 * _NS))
    tail = n_nodes - rq * _NS
    assert tail <= _CH and tail % 8 == 0
    return rq, tail


def _deg_kernel(dst3, n_nodes):
    nch = dst3.shape[1]
    rq, tail = _row_partition(n_nodes)
    mesh = plsc.VectorSubcoreMesh(core_axis_name="c", subcore_axis_name="s")

    @functools.partial(
        pl.kernel, mesh=mesh,
        out_type=jax.ShapeDtypeStruct((_NC, n_nodes, _LANES), jnp.float32),
        scratch_types=[
            pltpu.VMEM((nch, _CH), jnp.int32),
            pltpu.VMEM((_CH, _LANES), jnp.float32),
            pltpu.VMEM((_CH, _LANES), jnp.float32),
            pltpu.MemorySpace.VMEM_SHARED((n_nodes, _LANES), jnp.float32),
        ])
    def k(dst_hbm, out_hbm, didx, ones_b, zeros_b, bins):
        cid = lax.axis_index("c")
        sid = lax.axis_index("s")
        wid = sid * _NC + cid
        one = jnp.ones((_LANES,), jnp.float32)
        zero = jnp.zeros((_LANES,), jnp.float32)

        def fill(i, _):
            ones_b[i, :] = one
            zeros_b[i, :] = zero
            return 0
        lax.fori_loop(0, _CH, fill, 0)

        r0 = pl.multiple_of(sid * rq, 8)
        for st in range(0, rq, 96):
            ln = min(96, rq - st)
            pltpu.sync_copy(zeros_b.at[pl.ds(0, ln)],
                            bins.at[pl.ds(r0 + st, ln)])
        if tail:
            @pl.when(sid == 0)
            def _():
                pltpu.sync_copy(zeros_b.at[pl.ds(0, tail)],
                                bins.at[pl.ds(rq * _NS, tail)])
        plsc.subcore_barrier()

        pltpu.sync_copy(dst_hbm.at[wid], didx)

        def body(c, _):
            pltpu.sync_copy(ones_b, bins.at[didx.at[c]], add=True)
            return 0
        lax.fori_loop(0, nch, body, 0)
        plsc.subcore_barrier()

        pltpu.sync_copy(bins.at[pl.ds(r0, rq)],
                        out_hbm.at[cid].at[pl.ds(r0, rq)])
        if tail:
            @pl.when(sid == 0)
            def _():
                pltpu.sync_copy(bins.at[pl.ds(rq * _NS, tail)],
                                out_hbm.at[cid].at[pl.ds(rq * _NS, tail)])

    return k(dst3)


def _msg_kernel(hn, src3, dst3):
    n_nodes, d = hn.shape
    nch = src3.shape[1]
    rq, tail = _row_partition(n_nodes)
    mesh = plsc.VectorSubcoreMesh(core_axis_name="c", subcore_axis_name="s",
                                  num_cores=1)

    @functools.partial(
        pl.kernel, mesh=mesh,
        out_type=jax.ShapeDtypeStruct((1, n_nodes, d), jnp.float32),
        scratch_types=[
            pltpu.VMEM((2, _CH), jnp.int32),
            pltpu.VMEM((2, _CH), jnp.int32),
            pltpu.VMEM((_CH, d), jnp.float32),
            pltpu.VMEM((_CH, d), jnp.float32),
            pltpu.MemorySpace.VMEM_SHARED((n_nodes, d), jnp.float32),
            pltpu.SemaphoreType.DMA,
            pltpu.SemaphoreType.DMA,
            pltpu.SemaphoreType.DMA,
            pltpu.SemaphoreType.DMA,
        ])
    def k(hn_hbm, src_hbm, dst_hbm, out_hbm,
          sidx, didx, rows0, rows1, acc, gs0, gs1, is0, is1):
        sid = lax.axis_index("s")
        zero = jnp.zeros((_LANES,), jnp.float32)

        def zrow(i, _):
            def zcol(j, _):
                rows0[i, pl.ds(j * _LANES, _LANES)] = zero
                return 0
            return lax.fori_loop(0, d // _LANES, zcol, 0)
        lax.fori_loop(0, _CH, zrow, 0)

        r0 = pl.multiple_of(sid * rq, 8)
        for st in range(0, rq, 96):
            ln = min(96, rq - st)
            pltpu.sync_copy(rows0.at[pl.ds(0, ln)],
                            acc.at[pl.ds(r0 + st, ln)])
        if tail:
            @pl.when(sid == 0)
            def _():
                pltpu.sync_copy(rows0.at[pl.ds(0, tail)],
                                acc.at[pl.ds(rq * _NS, tail)])
        plsc.subcore_barrier()

        rows = (rows0, rows1)
        gsem = (gs0, gs1)
        isem = (is0, is1)

        def issue_idx(c, b):
            pltpu.async_copy(src_hbm.at[sid].at[c], sidx.at[b], isem[b])
            pltpu.async_copy(dst_hbm.at[sid].at[c], didx.at[b], isem[b])

        def wait_idx(b):
            pltpu.make_async_copy(src_hbm.at[sid].at[0], sidx.at[b],
                                  isem[b]).wait()
            pltpu.make_async_copy(dst_hbm.at[sid].at[0], didx.at[b],
                                  isem[b]).wait()

        def issue_gather(b):
            pltpu.async_copy(hn_hbm.at[sidx.at[b]], rows[b], gsem[b])

        def wait_gather(b):
            pltpu.make_async_copy(hn_hbm.at[sidx.at[b]], rows[b],
                                  gsem[b]).wait()

        issue_idx(0, 0)
        issue_idx(1, 1)
        wait_idx(0)
        issue_gather(0)

        def outer(o, _):
            for b in range(2):
                c = 2 * o + b
                b2 = 1 - b
                wait_gather(b)

                @pl.when(c + 1 < nch)
                def _():
                    wait_idx(b2)
                    issue_gather(b2)
                pltpu.sync_copy(rows[b], acc.at[didx.at[b]], add=True)

                @pl.when(c + 2 < nch)
                def _():
                    issue_idx(c + 2, b)
            return 0
        lax.fori_loop(0, nch // 2, outer, 0)
        plsc.subcore_barrier()

        pltpu.sync_copy(acc.at[pl.ds(r0, rq)],
                        out_hbm.at[0].at[pl.ds(r0, rq)])
        if tail:
            @pl.when(sid == 0)
            def _():
                pltpu.sync_copy(acc.at[pl.ds(rq * _NS, tail)],
                                out_hbm.at[0].at[pl.ds(rq * _NS, tail)])

    return k(hn, src3, dst3)


def _dinv_call(deg_parts, n_nodes):
    def body(p_ref, o_ref):
        s = p_ref[0] + p_ref[1]
        deg = s[:, 0:1] + 1.0
        o_ref[...] = jnp.broadcast_to(lax.rsqrt(deg), (n_nodes, 128))
    return pl.pallas_call(
        body, out_shape=jax.ShapeDtypeStruct((n_nodes, 128), jnp.float32),
    )(deg_parts)


def _scale_mm_call(xa, w, dinv_b, blk):
    n_nodes, _ = xa.shape
    d = w.shape[1]
    def body(x_ref, w_ref, dv_ref, o_ref):
        h = jnp.dot(x_ref[...], w_ref[...],
                    preferred_element_type=jnp.float32,
                    precision=lax.Precision.HIGHEST)
        o_ref[...] = h * dv_ref[...]
    return pl.pallas_call(
        body, grid=(n_nodes // blk,),
        in_specs=[pl.BlockSpec((blk, xa.shape[1]), lambda i: (i, 0)),
                  pl.BlockSpec(w.shape, lambda i: (0, 0)),
                  pl.BlockSpec((blk, 128), lambda i: (i, 0))],
        out_specs=pl.BlockSpec((blk, d), lambda i: (i, 0)),
        out_shape=jax.ShapeDtypeStruct((n_nodes, d), jnp.float32),
    )(xa, w, dinv_b)


def _combine_mm_call(acc, hn, dinv_b, b_row, w, blk):
    n_nodes, d = hn.shape
    def body(a_ref, hn_ref, dv_ref, b_ref, w_ref, o_ref):
        t = a_ref[0] + hn_ref[...]
        z = jnp.maximum(dv_ref[...] * t + b_ref[...], 0.0)
        h = jnp.dot(z, w_ref[...],
                    preferred_element_type=jnp.float32,
                    precision=lax.Precision.HIGHEST)
        o_ref[...] = h * dv_ref[...]
    return pl.pallas_call(
        body, grid=(n_nodes // blk,),
        in_specs=[pl.BlockSpec((1, blk, d), lambda i: (0, i, 0)),
                  pl.BlockSpec((blk, d), lambda i: (i, 0)),
                  pl.BlockSpec((blk, 128), lambda i: (i, 0)),
                  pl.BlockSpec((1, d), lambda i: (0, 0)),
                  pl.BlockSpec((d, d), lambda i: (0, 0))],
        out_specs=pl.BlockSpec((blk, d), lambda i: (i, 0)),
        out_shape=jax.ShapeDtypeStruct((n_nodes, d), jnp.float32),
    )(acc, hn, dinv_b, b_row, w)


def _final_call(acc, hn, dinv_b, b_row, batch3, wfc_p, bfc_row,
                n_graphs, blk):
    n_nodes, d = hn.shape
    def body(a_ref, hn_ref, dv_ref, b_ref, bt_ref, wf_ref, bf_ref, o_ref,
             sums, cnts):
        i = pl.program_id(0)

        @pl.when(i == 0)
        def _():
            sums[...] = jnp.zeros_like(sums)
            cnts[...] = jnp.zeros_like(cnts)

        t = a_ref[0] + hn_ref[...]
        z = jnp.maximum(dv_ref[...] * t + b_ref[...], 0.0)
        g = lax.broadcasted_iota(jnp.int32, (n_graphs, blk), 0)
        m = (bt_ref[0] == g).astype(jnp.float32)
        sums[...] += jnp.dot(m, z, preferred_element_type=jnp.float32,
                             precision=lax.Precision.HIGHEST)
        cnts[...] += jnp.sum(m, axis=1, keepdims=True)

        @pl.when(i == pl.num_programs(0) - 1)
        def _():
            pooled = sums[...] / jnp.maximum(cnts[...], 1.0)
            o_ref[...] = jnp.dot(pooled, wf_ref[...],
                                 preferred_element_type=jnp.float32,
                                 precision=lax.Precision.HIGHEST) + bf_ref[...]

    return pl.pallas_call(
        body, grid=(n_nodes // blk,),
        in_specs=[pl.BlockSpec((1, blk, d), lambda i: (0, i, 0)),
                  pl.BlockSpec((blk, d), lambda i: (i, 0)),
                  pl.BlockSpec((blk, 128), lambda i: (i, 0)),
                  pl.BlockSpec((1, d), lambda i: (0, 0)),
                  pl.BlockSpec((1, 1, blk), lambda i: (i, 0, 0)),
                  pl.BlockSpec((d, 128), lambda i: (0, 0)),
                  pl.BlockSpec((1, 128), lambda i: (0, 0))],
        out_specs=pl.BlockSpec((n_graphs, 128), lambda i: (0, 0)),
        out_shape=jax.ShapeDtypeStruct((n_graphs, 128), jnp.float32),
        scratch_shapes=[pltpu.VMEM((n_graphs, 128), jnp.float32),
                        pltpu.VMEM((n_graphs, 128), jnp.float32)],
    )(acc, hn, dinv_b, b_row, batch3, wfc_p, bfc_row)


def kernel(x, edge_index, batch, W1, b1, W2, b2, Wfc, bfc):
    n_nodes, _ = x.shape
    n_edges = edge_index.shape[1]
    d_hid = W1.shape[1]
    d_out = Wfc.shape[1]
    n_graphs = 16
    blk = 1000
    nch_deg = n_edges // (_NW * _CH)
    nch_msg = n_edges // (_NS * _CH)
    assert n_edges == _NW * _CH * nch_deg and n_nodes % blk == 0

    src3 = edge_index[0].reshape(_NS, nch_msg, _CH)
    dst3 = edge_index[1].reshape(_NS, nch_msg, _CH)
    dst3_deg = edge_index[1].reshape(_NW, nch_deg, _CH)

    deg_parts = _deg_kernel(dst3_deg, n_nodes)
    dinv_b = _dinv_call(deg_parts, n_nodes)

    hn1 = _scale_mm_call(x, W1, dinv_b, blk)
    acc1 = _msg_kernel(hn1, src3, dst3)
    hn2 = _combine_mm_call(acc1, hn1, dinv_b, b1.reshape(1, -1), W2, blk)
    acc2 = _msg_kernel(hn2, src3, dst3)

    wfc_p = jnp.zeros((d_hid, 128), jnp.float32).at[:, :d_out].set(Wfc)
    bfc_row = jnp.zeros((1, 128), jnp.float32).at[0, :d_out].set(bfc)
    q_pad = _final_call(acc2, hn2, dinv_b, b2.reshape(1, -1),
                        batch.reshape(n_nodes // blk, 1, blk), wfc_p, bfc_row,
                        n_graphs, blk)
    return q_pad[:, :d_out]


# speedup vs baseline: 17.7370x; 1.0841x over previous
import functools

import jax
import jax.numpy as jnp
from jax import lax
from jax.experimental import pallas as pl
from jax.experimental.pallas import tpu as pltpu
from jax.experimental.pallas import tpu_sc as plsc

_NC = 2
_NS = 16
_NW = _NC * _NS
_CH = 125
_LANES = 16


def _row_partition(n_nodes):
    rq = 8 * (n_nodes // (8 * _NS))
    tail = n_nodes - rq * _NS
    assert tail <= _CH and tail % 8 == 0
    return rq, tail


def _deg_kernel(dst3, n_nodes):
    nch = dst3.shape[1]
    rq, tail = _row_partition(n_nodes)
    mesh = plsc.VectorSubcoreMesh(core_axis_name="c", subcore_axis_name="s")

    @functools.partial(
        pl.kernel, mesh=mesh,
        out_type=jax.ShapeDtypeStruct((_NC, n_nodes, _LANES), jnp.float32),
        scratch_types=[
            pltpu.VMEM((nch, _CH), jnp.int32),
            pltpu.VMEM((_CH, _LANES), jnp.float32),
            pltpu.VMEM((_CH, _LANES), jnp.float32),
            pltpu.MemorySpace.VMEM_SHARED((n_nodes, _LANES), jnp.float32),
        ])
    def k(dst_hbm, out_hbm, didx, ones_b, zeros_b, bins):
        cid = lax.axis_index("c")
        sid = lax.axis_index("s")
        wid = sid * _NC + cid
        one = jnp.ones((_LANES,), jnp.float32)
        zero = jnp.zeros((_LANES,), jnp.float32)

        def fill(i, _):
            ones_b[i, :] = one
            zeros_b[i, :] = zero
            return 0
        lax.fori_loop(0, _CH, fill, 0)

        r0 = pl.multiple_of(sid * rq, 8)
        for st in range(0, rq, 96):
            ln = min(96, rq - st)
            pltpu.sync_copy(zeros_b.at[pl.ds(0, ln)],
                            bins.at[pl.ds(r0 + st, ln)])
        if tail:
            @pl.when(sid == 0)
            def _():
                pltpu.sync_copy(zeros_b.at[pl.ds(0, tail)],
                                bins.at[pl.ds(rq * _NS, tail)])
        plsc.subcore_barrier()

        pltpu.sync_copy(dst_hbm.at[wid], didx)

        def body(c, _):
            pltpu.sync_copy(ones_b, bins.at[didx.at[c]], add=True)
            return 0
        lax.fori_loop(0, nch, body, 0)
        plsc.subcore_barrier()

        pltpu.sync_copy(bins.at[pl.ds(r0, rq)],
                        out_hbm.at[cid].at[pl.ds(r0, rq)])
        if tail:
            @pl.when(sid == 0)
            def _():
                pltpu.sync_copy(bins.at[pl.ds(rq * _NS, tail)],
                                out_hbm.at[cid].at[pl.ds(rq * _NS, tail)])

    return k(dst3)


def _msg_kernel(hn, src3, dst3):
    n_nodes, d = hn.shape
    nch = src3.shape[1]
    assert nch % 4 == 0
    rq, tail = _row_partition(n_nodes)
    mesh = plsc.VectorSubcoreMesh(core_axis_name="c", subcore_axis_name="s",
                                  num_cores=1)

    @functools.partial(
        pl.kernel, mesh=mesh,
        out_type=jax.ShapeDtypeStruct((1, n_nodes, d), jnp.float32),
        scratch_types=[
            pltpu.VMEM((4, _CH), jnp.int32),
            pltpu.VMEM((4, _CH), jnp.int32),
            pltpu.VMEM((_CH, d), jnp.float32),
            pltpu.VMEM((_CH, d), jnp.float32),
            pltpu.MemorySpace.VMEM_SHARED((n_nodes, d), jnp.float32),
            [pltpu.SemaphoreType.DMA] * 2,
            [pltpu.SemaphoreType.DMA] * 2,
            [pltpu.SemaphoreType.DMA] * 4,
        ])
    def k(hn_hbm, src_hbm, dst_hbm, out_hbm,
          sidx, didx, rows0, rows1, acc, gsem, ssem, isem):
        sid = lax.axis_index("s")
        zero = jnp.zeros((_LANES,), jnp.float32)

        def zrow(i, _):
            def zcol(j, _):
                rows0[i, pl.ds(j * _LANES, _LANES)] = zero
                return 0
            return lax.fori_loop(0, d // _LANES, zcol, 0)
        lax.fori_loop(0, _CH, zrow, 0)

        r0 = pl.multiple_of(sid * rq, 8)
        for st in range(0, rq, 96):
            ln = min(96, rq - st)
            pltpu.sync_copy(rows0.at[pl.ds(0, ln)],
                            acc.at[pl.ds(r0 + st, ln)])
        if tail:
            @pl.when(sid == 0)
            def _():
                pltpu.sync_copy(rows0.at[pl.ds(0, tail)],
                                acc.at[pl.ds(rq * _NS, tail)])
        plsc.subcore_barrier()

        rows = (rows0, rows1)

        def issue_idx(c, b):
            pltpu.async_copy(src_hbm.at[sid].at[c], sidx.at[b], isem[b])
            pltpu.async_copy(dst_hbm.at[sid].at[c], didx.at[b], isem[b])

        def wait_idx(b):
            pltpu.make_async_copy(src_hbm.at[sid].at[0], sidx.at[b],
                                  isem[b]).wait()
            pltpu.make_async_copy(dst_hbm.at[sid].at[0], didx.at[b],
                                  isem[b]).wait()

        def issue_gather(ib, b):
            pltpu.async_copy(hn_hbm.at[sidx.at[ib]], rows[b], gsem[b])

        def wait_gather(ib, b):
            pltpu.make_async_copy(hn_hbm.at[sidx.at[ib]], rows[b],
                                  gsem[b]).wait()

        def wait_scatter(ib, b):
            pltpu.make_async_copy(rows[b], acc.at[didx.at[ib]],
                                  ssem[b]).wait()

        issue_idx(0, 0)
        issue_idx(1, 1)
        issue_idx(2, 2)
        wait_idx(0)
        issue_gather(0, 0)

        def outer(q, _):
            for j in range(4):
                c = 4 * q + j
                b = j % 2
                wait_gather(j, b)
                pltpu.async_copy(rows[b], acc.at[didx.at[j]], ssem[b],
                                 add=True)

                @pl.when(c >= 1)
                def _():
                    wait_scatter((j - 1) % 4, 1 - b)

                @pl.when(c + 3 < nch)
                def _():
                    issue_idx(c + 3, (j + 3) % 4)

                @pl.when(c + 1 < nch)
                def _():
                    wait_idx((j + 1) % 4)
                    issue_gather((j + 1) % 4, 1 - b)
            return 0
        lax.fori_loop(0, nch // 4, outer, 0)
        wait_scatter(3, 1)
        plsc.subcore_barrier()

        pltpu.sync_copy(acc.at[pl.ds(r0, rq)],
                        out_hbm.at[0].at[pl.ds(r0, rq)])
        if tail:
            @pl.when(sid == 0)
            def _():
                pltpu.sync_copy(acc.at[pl.ds(rq * _NS, tail)],
                                out_hbm.at[0].at[pl.ds(rq * _NS, tail)])

    return k(hn, src3, dst3)


def _dinv_call(deg_parts, n_nodes):
    def body(p_ref, o_ref):
        s = p_ref[0] + p_ref[1]
        deg = s[:, 0:1] + 1.0
        o_ref[...] = jnp.broadcast_to(lax.rsqrt(deg), (n_nodes, 128))
    return pl.pallas_call(
        body, out_shape=jax.ShapeDtypeStruct((n_nodes, 128), jnp.float32),
    )(deg_parts)


def _scale_mm_call(xa, w, dinv_b, blk):
    n_nodes, _ = xa.shape
    d = w.shape[1]
    def body(x_ref, w_ref, dv_ref, o_ref):
        h = jnp.dot(x_ref[...], w_ref[...],
                    preferred_element_type=jnp.float32,
                    precision=lax.Precision.HIGHEST)
        o_ref[...] = h * dv_ref[...]
    return pl.pallas_call(
        body, grid=(n_nodes // blk,),
        in_specs=[pl.BlockSpec((blk, xa.shape[1]), lambda i: (i, 0)),
                  pl.BlockSpec(w.shape, lambda i: (0, 0)),
                  pl.BlockSpec((blk, 128), lambda i: (i, 0))],
        out_specs=pl.BlockSpec((blk, d), lambda i: (i, 0)),
        out_shape=jax.ShapeDtypeStruct((n_nodes, d), jnp.float32),
    )(xa, w, dinv_b)


def _combine_mm_call(acc, hn, dinv_b, b_row, w, blk):
    n_nodes, d = hn.shape
    def body(a_ref, hn_ref, dv_ref, b_ref, w_ref, o_ref):
        t = a_ref[0] + hn_ref[...]
        z = jnp.maximum(dv_ref[...] * t + b_ref[...], 0.0)
        h = jnp.dot(z, w_ref[...],
                    preferred_element_type=jnp.float32,
                    precision=lax.Precision.HIGHEST)
        o_ref[...] = h * dv_ref[...]
    return pl.pallas_call(
        body, grid=(n_nodes // blk,),
        in_specs=[pl.BlockSpec((1, blk, d), lambda i: (0, i, 0)),
                  pl.BlockSpec((blk, d), lambda i: (i, 0)),
                  pl.BlockSpec((blk, 128), lambda i: (i, 0)),
                  pl.BlockSpec((1, d), lambda i: (0, 0)),
                  pl.BlockSpec((d, d), lambda i: (0, 0))],
        out_specs=pl.BlockSpec((blk, d), lambda i: (i, 0)),
        out_shape=jax.ShapeDtypeStruct((n_nodes, d), jnp.float32),
    )(acc, hn, dinv_b, b_row, w)


def _final_call(acc, hn, dinv_b, b_row, batch3, wfc_p, bfc_row,
                n_graphs, blk):
    n_nodes, d = hn.shape
    def body(a_ref, hn_ref, dv_ref, b_ref, bt_ref, wf_ref, bf_ref, o_ref,
             sums, cnts):
        i = pl.program_id(0)

        @pl.when(i == 0)
        def _():
            sums[...] = jnp.zeros_like(sums)
            cnts[...] = jnp.zeros_like(cnts)

        t = a_ref[0] + hn_ref[...]
        z = jnp.maximum(dv_ref[...] * t + b_ref[...], 0.0)
        g = lax.broadcasted_iota(jnp.int32, (n_graphs, blk), 0)
        m = (bt_ref[0] == g).astype(jnp.float32)
        sums[...] += jnp.dot(m, z, preferred_element_type=jnp.float32,
                             precision=lax.Precision.HIGHEST)
        cnts[...] += jnp.sum(m, axis=1, keepdims=True)

        @pl.when(i == pl.num_programs(0) - 1)
        def _():
            pooled = sums[...] / jnp.maximum(cnts[...], 1.0)
            o_ref[...] = jnp.dot(pooled, wf_ref[...],
                                 preferred_element_type=jnp.float32,
                                 precision=lax.Precision.HIGHEST) + bf_ref[...]

    return pl.pallas_call(
        body, grid=(n_nodes // blk,),
        in_specs=[pl.BlockSpec((1, blk, d), lambda i: (0, i, 0)),
                  pl.BlockSpec((blk, d), lambda i: (i, 0)),
                  pl.BlockSpec((blk, 128), lambda i: (i, 0)),
                  pl.BlockSpec((1, d), lambda i: (0, 0)),
                  pl.BlockSpec((1, 1, blk), lambda i: (i, 0, 0)),
                  pl.BlockSpec((d, 128), lambda i: (0, 0)),
                  pl.BlockSpec((1, 128), lambda i: (0, 0))],
        out_specs=pl.BlockSpec((n_graphs, 128), lambda i: (0, 0)),
        out_shape=jax.ShapeDtypeStruct((n_graphs, 128), jnp.float32),
        scratch_shapes=[pltpu.VMEM((n_graphs, 128), jnp.float32),
                        pltpu.VMEM((n_graphs, 128), jnp.float32)],
    )(acc, hn, dinv_b, b_row, batch3, wfc_p, bfc_row)


def kernel(x, edge_index, batch, W1, b1, W2, b2, Wfc, bfc):
    n_nodes, _ = x.shape
    n_edges = edge_index.shape[1]
    d_hid = W1.shape[1]
    d_out = Wfc.shape[1]
    n_graphs = 16
    blk = 1000
    nch_deg = n_edges // (_NW * _CH)
    nch_msg = n_edges // (_NS * _CH)
    assert n_edges == _NW * _CH * nch_deg and n_nodes % blk == 0

    src3 = edge_index[0].reshape(_NS, nch_msg, _CH)
    dst3 = edge_index[1].reshape(_NS, nch_msg, _CH)
    dst3_deg = edge_index[1].reshape(_NW, nch_deg, _CH)

    deg_parts = _deg_kernel(dst3_deg, n_nodes)
    dinv_b = _dinv_call(deg_parts, n_nodes)

    hn1 = _scale_mm_call(x, W1, dinv_b, blk)
    acc1 = _msg_kernel(hn1, src3, dst3)
    hn2 = _combine_mm_call(acc1, hn1, dinv_b, b1.reshape(1, -1), W2, blk)
    acc2 = _msg_kernel(hn2, src3, dst3)

    wfc_p = jnp.zeros((d_hid, 128), jnp.float32).at[:, :d_out].set(Wfc)
    bfc_row = jnp.zeros((1, 128), jnp.float32).at[0, :d_out].set(bfc)
    q_pad = _final_call(acc2, hn2, dinv_b, b2.reshape(1, -1),
                        batch.reshape(n_nodes // blk, 1, blk), wfc_p, bfc_row,
                        n_graphs, blk)
    return q_pad[:, :d_out]


# speedup vs baseline: 17.8464x; 1.0062x over previous
import functools

import jax
import jax.numpy as jnp
from jax import lax
from jax.experimental import pallas as pl
from jax.experimental.pallas import tpu as pltpu
from jax.experimental.pallas import tpu_sc as plsc

_NC = 2
_NS = 16
_NW = _NC * _NS
_CH = 125
_LANES = 16


def _row_partition(n_nodes):
    rq = 8 * (n_nodes // (8 * _NS))
    tail = n_nodes - rq * _NS
    assert tail <= _CH and tail % 8 == 0
    return rq, tail


def _deg_kernel(dst3, n_nodes):
    nch = dst3.shape[1]
    rq, tail = _row_partition(n_nodes)
    mesh = plsc.VectorSubcoreMesh(core_axis_name="c", subcore_axis_name="s")

    @functools.partial(
        pl.kernel, mesh=mesh,
        out_type=jax.ShapeDtypeStruct((_NC, n_nodes, _LANES), jnp.float32),
        scratch_types=[
            pltpu.VMEM((nch, _CH), jnp.int32),
            pltpu.VMEM((_CH, _LANES), jnp.float32),
            pltpu.VMEM((_CH, _LANES), jnp.float32),
            pltpu.MemorySpace.VMEM_SHARED((n_nodes, _LANES), jnp.float32),
        ])
    def k(dst_hbm, out_hbm, didx, ones_b, zeros_b, bins):
        cid = lax.axis_index("c")
        sid = lax.axis_index("s")
        wid = sid * _NC + cid
        one = jnp.ones((_LANES,), jnp.float32)
        zero = jnp.zeros((_LANES,), jnp.float32)

        def fill(i, _):
            ones_b[i, :] = one
            zeros_b[i, :] = zero
            return 0
        lax.fori_loop(0, _CH, fill, 0)

        r0 = pl.multiple_of(sid * rq, 8)
        for st in range(0, rq, 96):
            ln = min(96, rq - st)
            pltpu.sync_copy(zeros_b.at[pl.ds(0, ln)],
                            bins.at[pl.ds(r0 + st, ln)])
        if tail:
            @pl.when(sid == 0)
            def _():
                pltpu.sync_copy(zeros_b.at[pl.ds(0, tail)],
                                bins.at[pl.ds(rq * _NS, tail)])
        plsc.subcore_barrier()

        pltpu.sync_copy(dst_hbm.at[wid], didx)

        def body(c, _):
            pltpu.sync_copy(ones_b, bins.at[didx.at[c]], add=True)
            return 0
        lax.fori_loop(0, nch, body, 0)
        plsc.subcore_barrier()

        pltpu.sync_copy(bins.at[pl.ds(r0, rq)],
                        out_hbm.at[cid].at[pl.ds(r0, rq)])
        if tail:
            @pl.when(sid == 0)
            def _():
                pltpu.sync_copy(bins.at[pl.ds(rq * _NS, tail)],
                                out_hbm.at[cid].at[pl.ds(rq * _NS, tail)])

    return k(dst3)


def _msg_kernel(hn, src3, dst3):
    n_nodes, d = hn.shape
    nch = src3.shape[1]
    assert nch % 4 == 0
    rq, tail = _row_partition(n_nodes)
    mesh = plsc.VectorSubcoreMesh(core_axis_name="c", subcore_axis_name="s",
                                  num_cores=1)

    @functools.partial(
        pl.kernel, mesh=mesh,
        out_type=jax.ShapeDtypeStruct((1, n_nodes, d), jnp.float32),
        scratch_types=[
            pltpu.VMEM((4, _CH), jnp.int32),
            pltpu.VMEM((4, _CH), jnp.int32),
            pltpu.VMEM((_CH, d), jnp.float32),
            pltpu.VMEM((_CH, d), jnp.float32),
            pltpu.MemorySpace.VMEM_SHARED((n_nodes, d), jnp.float32),
            [pltpu.SemaphoreType.DMA] * 2,
            [pltpu.SemaphoreType.DMA] * 2,
            [pltpu.SemaphoreType.DMA] * 4,
        ])
    def k(hn_hbm, src_hbm, dst_hbm, out_hbm,
          sidx, didx, rows0, rows1, acc, gsem, ssem, isem):
        sid = lax.axis_index("s")
        zero = jnp.zeros((_LANES,), jnp.float32)

        def zrow(i, _):
            def zcol(j, _):
                rows0[i, pl.ds(j * _LANES, _LANES)] = zero
                return 0
            return lax.fori_loop(0, d // _LANES, zcol, 0)
        lax.fori_loop(0, _CH, zrow, 0)

        r0 = pl.multiple_of(sid * rq, 8)
        for st in range(0, rq, 96):
            ln = min(96, rq - st)
            pltpu.sync_copy(rows0.at[pl.ds(0, ln)],
                            acc.at[pl.ds(r0 + st, ln)])
        if tail:
            @pl.when(sid == 0)
            def _():
                pltpu.sync_copy(rows0.at[pl.ds(0, tail)],
                                acc.at[pl.ds(rq * _NS, tail)])
        plsc.subcore_barrier()

        rows = (rows0, rows1)

        def issue_idx(c, b):
            pltpu.async_copy(src_hbm.at[sid].at[c], sidx.at[b], isem[b])
            pltpu.async_copy(dst_hbm.at[sid].at[c], didx.at[b], isem[b])

        def wait_idx(b):
            pltpu.make_async_copy(src_hbm.at[sid].at[0], sidx.at[b],
                                  isem[b]).wait()
            pltpu.make_async_copy(dst_hbm.at[sid].at[0], didx.at[b],
                                  isem[b]).wait()

        def issue_gather(ib, b):
            pltpu.async_copy(hn_hbm.at[sidx.at[ib]], rows[b], gsem[b])

        def wait_gather(ib, b):
            pltpu.make_async_copy(hn_hbm.at[sidx.at[ib]], rows[b],
                                  gsem[b]).wait()

        def wait_scatter(ib, b):
            pltpu.make_async_copy(rows[b], acc.at[didx.at[ib]],
                                  ssem[b]).wait()

        issue_idx(0, 0)
        issue_idx(1, 1)
        issue_idx(2, 2)
        wait_idx(0)
        issue_gather(0, 0)

        def outer(q, _):
            for j in range(4):
                c = 4 * q + j
                b = j % 2
                wait_gather(j, b)
                pltpu.async_copy(rows[b], acc.at[didx.at[j]], ssem[b],
                                 add=True)

                @pl.when(c >= 1)
                def _():
                    wait_scatter((j - 1) % 4, 1 - b)

                @pl.when(c + 3 < nch)
                def _():
                    issue_idx(c + 3, (j + 3) % 4)

                @pl.when(c + 1 < nch)
                def _():
                    wait_idx((j + 1) % 4)
                    issue_gather((j + 1) % 4, 1 - b)
            return 0
        lax.fori_loop(0, nch // 4, outer, 0)
        wait_scatter(3, 1)
        plsc.subcore_barrier()

        pltpu.sync_copy(acc.at[pl.ds(r0, rq)],
                        out_hbm.at[0].at[pl.ds(r0, rq)])
        if tail:
            @pl.when(sid == 0)
            def _():
                pltpu.sync_copy(acc.at[pl.ds(rq * _NS, tail)],
                                out_hbm.at[0].at[pl.ds(rq * _NS, tail)])

    return k(hn, src3, dst3)


def _dinv_blk(p_ref):
    deg = p_ref[0][:, 0:1] + p_ref[1][:, 0:1] + 1.0
    return lax.rsqrt(deg)


def _scale_mm_call(xa, w, deg_parts, blk):
    n_nodes, _ = xa.shape
    d = w.shape[1]
    def body(x_ref, w_ref, p_ref, o_ref):
        h = jnp.dot(x_ref[...], w_ref[...],
                    preferred_element_type=jnp.float32,
                    precision=lax.Precision.HIGHEST)
        o_ref[...] = h * _dinv_blk(p_ref)
    return pl.pallas_call(
        body, grid=(n_nodes // blk,),
        in_specs=[pl.BlockSpec((blk, xa.shape[1]), lambda i: (i, 0)),
                  pl.BlockSpec(w.shape, lambda i: (0, 0)),
                  pl.BlockSpec((2, blk, _LANES), lambda i: (0, i, 0))],
        out_specs=pl.BlockSpec((blk, d), lambda i: (i, 0)),
        out_shape=jax.ShapeDtypeStruct((n_nodes, d), jnp.float32),
    )(xa, w, deg_parts)


def _combine_mm_call(acc, hn, deg_parts, b_row, w, blk):
    n_nodes, d = hn.shape
    def body(a_ref, hn_ref, p_ref, b_ref, w_ref, o_ref):
        dinv = _dinv_blk(p_ref)
        t = a_ref[0] + hn_ref[...]
        z = jnp.maximum(dinv * t + b_ref[...], 0.0)
        h = jnp.dot(z, w_ref[...],
                    preferred_element_type=jnp.float32,
                    precision=lax.Precision.HIGHEST)
        o_ref[...] = h * dinv
    return pl.pallas_call(
        body, grid=(n_nodes // blk,),
        in_specs=[pl.BlockSpec((1, blk, d), lambda i: (0, i, 0)),
                  pl.BlockSpec((blk, d), lambda i: (i, 0)),
                  pl.BlockSpec((2, blk, _LANES), lambda i: (0, i, 0)),
                  pl.BlockSpec((1, d), lambda i: (0, 0)),
                  pl.BlockSpec((d, d), lambda i: (0, 0))],
        out_specs=pl.BlockSpec((blk, d), lambda i: (i, 0)),
        out_shape=jax.ShapeDtypeStruct((n_nodes, d), jnp.float32),
    )(acc, hn, deg_parts, b_row, w)


def _final_call(acc, hn, deg_parts, b_row, batch3, wfc_p, bfc_row,
                n_graphs, blk):
    n_nodes, d = hn.shape
    def body(a_ref, hn_ref, p_ref, b_ref, bt_ref, wf_ref, bf_ref, o_ref,
             sums, cnts):
        i = pl.program_id(0)

        @pl.when(i == 0)
        def _():
            sums[...] = jnp.zeros_like(sums)
            cnts[...] = jnp.zeros_like(cnts)

        t = a_ref[0] + hn_ref[...]
        z = jnp.maximum(_dinv_blk(p_ref) * t + b_ref[...], 0.0)
        g = lax.broadcasted_iota(jnp.int32, (n_graphs, blk), 0)
        m = (bt_ref[0] == g).astype(jnp.float32)
        sums[...] += jnp.dot(m, z, preferred_element_type=jnp.float32,
                             precision=lax.Precision.HIGHEST)
        cnts[...] += jnp.sum(m, axis=1, keepdims=True)

        @pl.when(i == pl.num_programs(0) - 1)
        def _():
            pooled = sums[...] / jnp.maximum(cnts[...], 1.0)
            o_ref[...] = jnp.dot(pooled, wf_ref[...],
                                 preferred_element_type=jnp.float32,
                                 precision=lax.Precision.HIGHEST) + bf_ref[...]

    return pl.pallas_call(
        body, grid=(n_nodes // blk,),
        in_specs=[pl.BlockSpec((1, blk, d), lambda i: (0, i, 0)),
                  pl.BlockSpec((blk, d), lambda i: (i, 0)),
                  pl.BlockSpec((2, blk, _LANES), lambda i: (0, i, 0)),
                  pl.BlockSpec((1, d), lambda i: (0, 0)),
                  pl.BlockSpec((1, 1, blk), lambda i: (i, 0, 0)),
                  pl.BlockSpec((d, 128), lambda i: (0, 0)),
                  pl.BlockSpec((1, 128), lambda i: (0, 0))],
        out_specs=pl.BlockSpec((n_graphs, 128), lambda i: (0, 0)),
        out_shape=jax.ShapeDtypeStruct((n_graphs, 128), jnp.float32),
        scratch_shapes=[pltpu.VMEM((n_graphs, 128), jnp.float32),
                        pltpu.VMEM((n_graphs, 128), jnp.float32)],
    )(acc, hn, deg_parts, b_row, batch3, wfc_p, bfc_row)


def kernel(x, edge_index, batch, W1, b1, W2, b2, Wfc, bfc):
    n_nodes, _ = x.shape
    n_edges = edge_index.shape[1]
    d_hid = W1.shape[1]
    d_out = Wfc.shape[1]
    n_graphs = 16
    blk = 1000
    nch_deg = n_edges // (_NW * _CH)
    nch_msg = n_edges // (_NS * _CH)
    assert n_edges == _NW * _CH * nch_deg and n_nodes % blk == 0

    src3 = edge_index[0].reshape(_NS, nch_msg, _CH)
    dst3 = edge_index[1].reshape(_NS, nch_msg, _CH)
    dst3_deg = edge_index[1].reshape(_NW, nch_deg, _CH)

    deg_parts = _deg_kernel(dst3_deg, n_nodes)

    hn1 = _scale_mm_call(x, W1, deg_parts, blk)
    acc1 = _msg_kernel(hn1, src3, dst3)
    hn2 = _combine_mm_call(acc1, hn1, deg_parts, b1.reshape(1, -1), W2, blk)
    acc2 = _msg_kernel(hn2, src3, dst3)

    wfc_p = jnp.zeros((d_hid, 128), jnp.float32).at[:, :d_out].set(Wfc)
    bfc_row = jnp.zeros((1, 128), jnp.float32).at[0, :d_out].set(bfc)
    q_pad = _final_call(acc2, hn2, deg_parts, b2.reshape(1, -1),
                        batch.reshape(n_nodes // blk, 1, blk), wfc_p, bfc_row,
                        n_graphs, blk)
    return q_pad[:, :d_out]
